# baseline (speedup 1.0000x reference)
import functools

import jax
import jax.numpy as jnp
from jax import lax
from jax.experimental import pallas as pl
from jax.experimental.pallas import tpu as pltpu
from jax.experimental.pallas import tpu_sc as plsc

VOCAB = 100000
EMB_DIM = 64
BATCH = 16384

_info = plsc.get_sparse_core_info()
_NC = _info.num_cores
_NS = _info.num_subcores
_NW = _NC * _NS
_RANGE = 3200
_PASSW = 512
_NPASS = -(-_RANGE // _PASSW)
_FETCHW = 512
_SA_MAX = ((VOCAB - _FETCHW) // 128) * 128
_TAILC = ((VOCAB // 128) * 128)
_TAILW = VOCAB - _TAILC
_PCAP = 128
_TRASH = BATCH
_IROWS = BATCH + _PCAP

_mesh = plsc.VectorSubcoreMesh(core_axis_name="c", subcore_axis_name="s")


@functools.partial(
    pl.kernel,
    mesh=_mesh,
    compiler_params=pltpu.CompilerParams(
        use_tc_tiling_on_sc=True, needs_layout_passes=False),
    out_type=jax.ShapeDtypeStruct((_IROWS, 128), jnp.float32),
    scratch_types=[
        pltpu.VMEM((BATCH,), jnp.int32),
        pltpu.VMEM((BATCH + 16,), jnp.int32),
        pltpu.VMEM((_PCAP + 16,), jnp.int32),
        pltpu.VMEM((_PCAP,), jnp.int32),
        pltpu.VMEM((64, _FETCHW), jnp.float32),
        pltpu.VMEM((64, _FETCHW), jnp.float32),
        pltpu.VMEM((64, _TAILW), jnp.float32),
        pltpu.VMEM((_PCAP, 128), jnp.float32),
        pltpu.SemaphoreType.DMA,
        pltpu.SemaphoreType.DMA,
    ],
)
def _gather_kernel(idx_hbm, tT_hbm, tail_hbm, out_hbm, idx_v, pos_v, ppos_v,
                   dsti_v, stripe0, stripe1, tail_v, stage_v, sem_s, sem_w):
    wid = lax.axis_index("s") * _NC + lax.axis_index("c")
    lo = wid * _RANGE
    hi = jnp.minimum(lo + _RANGE, VOCAB)
    iota16 = lax.iota(jnp.int32, 16)

    pltpu.sync_copy(idx_hbm, idx_v)
    pltpu.sync_copy(tail_hbm, tail_v)

    def scan_g(g, cnt):
        v = idx_v[pl.ds(g * 16, 16)]
        m = (v >= lo) & (v < hi)
        plsc.store_compressed(pos_v.at[pl.ds(cnt, 16)], iota16 + g * 16, mask=m)
        return cnt + plsc.all_reduce_population_count(m)[0]

    n = lax.fori_loop(0, BATCH // 16, scan_g, jnp.int32(0))
    ngrp = (n + 15) // 16

    stripes = [stripe0, stripe1]

    def fire(p):
        c0 = lo + p * _PASSW
        sa = jnp.minimum(c0, _SA_MAX)
        return sa, pltpu.async_copy(
            tT_hbm.at[:, pl.ds(sa, _FETCHW)], stripes[p % 2], sem_s)

    sa_cur, pending = fire(0)
    pend_w = jnp.int32(0)

    for p in range(_NPASS + 1):
        if p < _NPASS:
            c0 = lo + p * _PASSW
            cend = jnp.minimum(jnp.minimum(c0 + _PASSW, hi),
                               jnp.int32(_TAILC))
            stripe = stripes[p % 2]
            fw = _FETCHW
        else:
            c0 = jnp.int32(_TAILC)
            cend = hi
            stripe = tail_v
            sa_cur = jnp.int32(_TAILC)
            fw = _TAILW
        nxt = fire(p + 1) if p + 1 < _NPASS else None
        if p < _NPASS:
            pending.wait()

        def round_body(r, carry, c0=c0, cend=cend, sa=sa_cur, stripe=stripe,
                       fw=fw):
            pend_w, _total = carry

            def cmp_g(g, cc, c0=c0, cend=cend):
                pcnt, ordn = cc
                lanev = (iota16 + g * 16) < n
                pv = pos_v[pl.ds(g * 16, 16)]
                val = plsc.load_gather(idx_v, [pv & (BATCH - 1)])
                m = lanev & (val >= c0) & (val < cend)
                mi = m.astype(jnp.int32)
                excl = ordn + plsc.cumsum(mi) - mi
                sel = m & (excl >= r * _PCAP) & (excl < r * _PCAP + _PCAP)
                plsc.store_compressed(ppos_v.at[pl.ds(pcnt, 16)], pv, mask=sel)
                return (pcnt + plsc.all_reduce_population_count(sel)[0],
                        ordn + plsc.all_reduce_population_count(m)[0])

            pcnt, total = lax.fori_loop(0, ngrp, cmp_g,
                                        (jnp.int32(0), jnp.int32(0)))

            @pl.when(pend_w == 1)
            def _():
                pltpu.make_async_copy(
                    out_hbm.at[pl.ds(0, _PCAP)], stage_v, sem_w).wait()

            for q in range(_PCAP // 16):
                dsti_v[pl.ds(q * 16, 16)] = _TRASH + q * 16 + iota16

            def ext_g(q, _, sa=sa, stripe=stripe, fw=fw):
                lv = (iota16 + q * 16) < pcnt
                pp = ppos_v[pl.ds(q * 16, 16)]
                val = plsc.load_gather(idx_v, [pp & (BATCH - 1)])
                vrel = jnp.clip(val - sa, 0, fw - 1)
                dsti_v[pl.ds(q * 16, 16)] = jnp.where(
                    lv, pp, _TRASH + q * 16 + iota16)
                row16 = q * 16 + iota16
                for d in range(EMB_DIM):
                    d16 = jnp.full((16,), d, jnp.int32)
                    vals = plsc.load_gather(stripe, [d16, vrel])
                    plsc.store_scatter(stage_v, [row16, d16], vals)
                return 0

            lax.fori_loop(0, (pcnt + 15) // 16, ext_g, 0)

            @pl.when(pcnt > 0)
            def _():
                pltpu.async_copy(stage_v, out_hbm.at[dsti_v], sem_w)

            return (jnp.where(pcnt > 0, 1, 0), total)

        pend_w, total = round_body(jnp.int32(0), (pend_w, jnp.int32(0)))
        extra = (total + _PCAP - 1) // _PCAP
        pend_w, _ = lax.fori_loop(1, extra, round_body, (pend_w, total))

        if nxt is not None:
            sa_cur, pending = nxt

    @pl.when(pend_w == 1)
    def _():
        pltpu.make_async_copy(
            out_hbm.at[pl.ds(0, _PCAP)], stage_v, sem_w).wait()


def kernel(token_ids, embedding_weight):
    interm = _gather_kernel(token_ids.astype(jnp.int32), embedding_weight.T,
                            embedding_weight[_TAILC:, :].T)
    return interm[:BATCH, :EMB_DIM]
